# baseline (speedup 1.0000x reference)
import functools

import jax
import jax.numpy as jnp
from jax import lax
from jax.experimental import pallas as pl
from jax.experimental.pallas import tpu as pltpu
from jax.experimental.pallas import tpu_sc as plsc

NUM_HEADS = 8
IN_F = 256
OUT_F = 256
PER_HEAD = 32
N = 10000
E = 160000
HALF = 128
HPC = 4

NC = 2
NS = 16
EPW = E // NS
CHUNK = 80
NCHUNK = EPW // CHUNK
NPAIR = (NCHUNK - 1) // 2
N_PAD = 10240
ROWS_PER_SUB = N_PAD // NS

_mesh = plsc.VectorSubcoreMesh(core_axis_name="c", subcore_axis_name="s")
_params = pltpu.CompilerParams(needs_layout_passes=False)


def _t2_body(x_ref, w_ref, o_ref):
    o_ref[...] = jnp.dot(x_ref[...], w_ref[...],
                         preferred_element_type=jnp.float32)


def _compute_t2(x, wcat_t):
    rt = 1000
    return pl.pallas_call(
        _t2_body,
        grid=(NC, N // rt),
        in_specs=[
            pl.BlockSpec((rt, IN_F), lambda h, r: (r, 0)),
            pl.BlockSpec((IN_F, HALF), lambda h, r: (0, h)),
        ],
        out_specs=pl.BlockSpec((rt, HALF), lambda h, r: (h * (N // rt) + r, 0)),
        out_shape=jax.ShapeDtypeStruct((NC * N, HALF), jnp.float32),
    )(x, wcat_t)


def _scores_body(src_hbm, dst_hbm, t2_hbm, out_hbm,
                 sidx, didx, srowsA, drowsA, srowsB, drowsB, sch,
                 semAs, semAd, semBs, semBd):
    c = lax.axis_index("c")
    s = lax.axis_index("s")
    ibase = pl.multiple_of(c * E + s * EPW, 8)
    pltpu.sync_copy(src_hbm.at[pl.ds(ibase, EPW)], sidx)
    pltpu.sync_copy(dst_hbm.at[pl.ds(ibase, EPW)], didx)

    def start(i, srows, drows, sem_s, sem_d):
        sl = pl.ds(pl.multiple_of(i * CHUNK, 8), CHUNK)
        pltpu.async_copy(t2_hbm.at[sidx.at[sl]], srows, sem_s)
        pltpu.async_copy(t2_hbm.at[didx.at[sl]], drows, sem_d)

    def wait(i, srows, drows, sem_s, sem_d):
        sl = pl.ds(pl.multiple_of(i * CHUNK, 8), CHUNK)
        pltpu.make_async_copy(t2_hbm.at[sidx.at[sl]], srows, sem_s).wait()
        pltpu.make_async_copy(t2_hbm.at[didx.at[sl]], drows, sem_d).wait()

    def compute(i, srows, drows):
        def group(g, carry):
            iota = lax.broadcasted_iota(jnp.int32, (16,), 0)
            rid = iota + g * 16
            for h in range(HPC):
                acc = jnp.zeros((16,), jnp.float32)
                for cc in range(PER_HEAD):
                    col = h * PER_HEAD + ((cc + iota) & (PER_HEAD - 1))
                    sv = plsc.load_gather(srows, [rid, col])
                    dv = plsc.load_gather(drows, [rid, col])
                    acc = acc + sv * dv
                sch[pl.ds(h * EPW + i * CHUNK + g * 16, 16)] = acc
            return carry

        lax.fori_loop(0, CHUNK // 16, group, 0)

    start(0, srowsA, drowsA, semAs, semAd)

    def pair(ii, carry):
        i0 = ii * 2
        i1 = i0 + 1
        start(i1, srowsB, drowsB, semBs, semBd)
        wait(i0, srowsA, drowsA, semAs, semAd)
        compute(i0, srowsA, drowsA)
        start(i0 + 2, srowsA, drowsA, semAs, semAd)
        wait(i1, srowsB, drowsB, semBs, semBd)
        compute(i1, srowsB, drowsB)
        return carry

    lax.fori_loop(0, NPAIR, pair, 0)
    wait(NCHUNK - 1, srowsA, drowsA, semAs, semAd)
    compute(NCHUNK - 1, srowsA, drowsA)

    for h in range(HPC):
        hoff = pl.multiple_of((c * HPC + h) * E + s * EPW, 8)
        pltpu.sync_copy(sch.at[pl.ds(h * EPW, EPW)],
                        out_hbm.at[pl.ds(hoff, EPW)])


_sc_scores = functools.partial(
    pl.kernel,
    out_type=jax.ShapeDtypeStruct((NUM_HEADS * E,), jnp.float32),
    mesh=_mesh,
    compiler_params=_params,
    scratch_types=[
        pltpu.VMEM((EPW,), jnp.int32),
        pltpu.VMEM((EPW,), jnp.int32),
        pltpu.VMEM((CHUNK, HALF), jnp.float32),
        pltpu.VMEM((CHUNK, HALF), jnp.float32),
        pltpu.VMEM((CHUNK, HALF), jnp.float32),
        pltpu.VMEM((CHUNK, HALF), jnp.float32),
        pltpu.VMEM((HPC * EPW,), jnp.float32),
        pltpu.SemaphoreType.DMA,
        pltpu.SemaphoreType.DMA,
        pltpu.SemaphoreType.DMA,
        pltpu.SemaphoreType.DMA,
    ],
)(_scores_body)


def _softmax_body(s_ref, o_ref):
    sc = s_ref[...]
    m = jnp.max(sc, axis=1, keepdims=True)
    ex = jnp.exp(sc - m)
    z = jnp.sum(ex, axis=1, keepdims=True)
    o_ref[...] = ex / z


def _softmax(scores):
    return pl.pallas_call(
        _softmax_body,
        out_shape=jax.ShapeDtypeStruct((NUM_HEADS, E), jnp.float32),
    )(scores)


def _scatter_body(src_hbm, dst_hbm, t2_hbm, w_hbm, zeros_hbm, out_hbm,
                  sidx, didx, wchA, wchB, drowsA, drowsB, out_sh,
                  semA, semB, semSA, semSB):
    c = lax.axis_index("c")
    s = lax.axis_index("s")
    ebase = pl.multiple_of(s * EPW, 8)
    pltpu.sync_copy(src_hbm.at[pl.ds(ebase, EPW)], sidx)
    dbase = pl.multiple_of(c * E + s * EPW, 8)
    pltpu.sync_copy(dst_hbm.at[pl.ds(dbase, EPW)], didx)

    rbase = pl.multiple_of(s * ROWS_PER_SUB, 8)
    pltpu.sync_copy(zeros_hbm.at[pl.ds(rbase, ROWS_PER_SUB)],
                    out_sh.at[pl.ds(rbase, ROWS_PER_SUB)])
    plsc.subcore_barrier()

    def start(i, drows, wch, sem):
        sl = pl.ds(pl.multiple_of(i * CHUNK, 8), CHUNK)
        pltpu.async_copy(t2_hbm.at[didx.at[sl]], drows, sem)
        woff = pl.multiple_of((c * E + s * EPW + i * CHUNK) * HPC, 8)
        pltpu.async_copy(w_hbm.at[pl.ds(woff, CHUNK * HPC)], wch, sem)

    def wait(i, drows, wch, sem):
        sl = pl.ds(pl.multiple_of(i * CHUNK, 8), CHUNK)
        pltpu.make_async_copy(t2_hbm.at[didx.at[sl]], drows, sem).wait()
        woff = pl.multiple_of((c * E + s * EPW + i * CHUNK) * HPC, 8)
        pltpu.make_async_copy(w_hbm.at[pl.ds(woff, CHUNK * HPC)], wch,
                              sem).wait()

    def weight(i, drows, wch):
        def group(g, carry):
            iota = lax.broadcasted_iota(jnp.int32, (16,), 0)
            rid = iota + g * 16
            rid4 = rid * HPC
            for h in range(HPC):
                wv = plsc.load_gather(wch, [rid4 + h])
                for cc in range(PER_HEAD):
                    col = h * PER_HEAD + ((cc + iota) & (PER_HEAD - 1))
                    v = plsc.load_gather(drows, [rid, col])
                    plsc.store_scatter(drows, [rid, col], v * wv)
            return carry

        lax.fori_loop(0, CHUNK // 16, group, 0)

    def start_scatter(i, drows, sem):
        return
        for k in range(CHUNK // 16):
            idxv = sidx[pl.ds(i * CHUNK + k * 16, 16)]
            pltpu.async_copy(drows.at[pl.ds(k * 16, 16)],
                             out_sh.at[idxv], sem, add=True)

    def wait_scatter(i, drows, sem):
        return
        for k in range(CHUNK // 16):
            idxv = sidx[pl.ds(i * CHUNK + k * 16, 16)]
            pltpu.make_async_copy(drows.at[pl.ds(k * 16, 16)],
                                  out_sh.at[idxv], sem).wait()

    start(0, drowsA, wchA, semA)

    def pair(ii, carry):
        i0 = ii * 2
        i1 = i0 + 1
        start(i1, drowsB, wchB, semB)
        wait(i0, drowsA, wchA, semA)
        weight(i0, drowsA, wchA)
        start_scatter(i0, drowsA, semSA)
        wait(i1, drowsB, wchB, semB)
        weight(i1, drowsB, wchB)
        start_scatter(i1, drowsB, semSB)
        wait_scatter(i0, drowsA, semSA)
        start(i0 + 2, drowsA, wchA, semA)
        wait_scatter(i1, drowsB, semSB)
        return carry

    lax.fori_loop(0, NPAIR, pair, 0)
    i_last = NCHUNK - 1
    wait(i_last, drowsA, wchA, semA)
    weight(i_last, drowsA, wchA)
    start_scatter(i_last, drowsA, semSA)
    wait_scatter(i_last, drowsA, semSA)

    plsc.subcore_barrier()
    obase = pl.multiple_of(c * N_PAD + s * ROWS_PER_SUB, 8)
    pltpu.sync_copy(out_sh.at[pl.ds(rbase, ROWS_PER_SUB)],
                    out_hbm.at[pl.ds(obase, ROWS_PER_SUB)])


_sc_scatter = functools.partial(
    pl.kernel,
    out_type=jax.ShapeDtypeStruct((NC * N_PAD, HALF), jnp.float32),
    mesh=_mesh,
    compiler_params=_params,
    scratch_types=[
        pltpu.VMEM((EPW,), jnp.int32),
        pltpu.VMEM((EPW,), jnp.int32),
        pltpu.VMEM((CHUNK * HPC,), jnp.float32),
        pltpu.VMEM((CHUNK * HPC,), jnp.float32),
        pltpu.VMEM((CHUNK, HALF), jnp.float32),
        pltpu.VMEM((CHUNK, HALF), jnp.float32),
        pltpu.VMEM_SHARED((N_PAD, HALF), jnp.float32),
        pltpu.SemaphoreType.DMA,
        pltpu.SemaphoreType.DMA,
        pltpu.SemaphoreType.DMA,
        pltpu.SemaphoreType.DMA,
    ],
)(_scatter_body)


def _merge_body(a0_ref, a1_ref, w0_ref, w1_ref, b_ref, o_ref):
    o_ref[...] = (jnp.dot(a0_ref[...], w0_ref[...],
                          preferred_element_type=jnp.float32)
                  + jnp.dot(a1_ref[...], w1_ref[...],
                            preferred_element_type=jnp.float32)
                  + b_ref[...])


def _merge(o0, o1, w0, w1, bm2):
    rt = 1000
    return pl.pallas_call(
        _merge_body,
        grid=(N // rt,),
        in_specs=[
            pl.BlockSpec((rt, HALF), lambda r: (r, 0)),
            pl.BlockSpec((rt, HALF), lambda r: (r, 0)),
            pl.BlockSpec((HALF, OUT_F), lambda r: (0, 0)),
            pl.BlockSpec((HALF, OUT_F), lambda r: (0, 0)),
            pl.BlockSpec((1, OUT_F), lambda r: (0, 0)),
        ],
        out_specs=pl.BlockSpec((rt, OUT_F), lambda r: (r, 0)),
        out_shape=jax.ShapeDtypeStruct((N, OUT_F), jnp.float32),
    )(o0, o1, w0, w1, bm2)


def kernel(node_features, edge_index, W, Wm, bm):
    src = edge_index[0]
    dst = edge_index[1]
    src2 = jnp.concatenate([src, src + N])
    dst2 = jnp.concatenate([dst, dst + N])
    wcat_t = W.reshape(OUT_F, IN_F).T
    t2 = _compute_t2(node_features, wcat_t)
    scores = _sc_scores(src2, dst2, t2)
    w = _softmax(scores.reshape(NUM_HEADS, E))
    zeros = jnp.zeros((N_PAD, HALF), jnp.float32)
    w_em = w.reshape(NC, HPC, E).transpose(0, 2, 1).reshape(-1)
    out2 = _sc_scatter(src, dst2, t2, w_em, zeros)
    w0 = Wm[:, :HALF].T
    w1 = Wm[:, HALF:].T
    return _merge(out2[:N], out2[N_PAD:N_PAD + N], w0, w1,
                  bm.reshape(1, OUT_F))


# speedup vs baseline: 1.2851x; 1.2851x over previous
import functools

import jax
import jax.numpy as jnp
from jax import lax
from jax.experimental import pallas as pl
from jax.experimental.pallas import tpu as pltpu
from jax.experimental.pallas import tpu_sc as plsc

NUM_HEADS = 8
IN_F = 256
OUT_F = 256
PER_HEAD = 32
N = 10000
E = 160000
HALF = 128
HPC = 4

NC = 2
NS = 16
EPW = E // NS
CHUNK = 80
NCHUNK = EPW // CHUNK
NPAIR = (NCHUNK - 1) // 2
N_PAD = 10240
ROWS_PER_SUB = N_PAD // NS

_mesh = plsc.VectorSubcoreMesh(core_axis_name="c", subcore_axis_name="s")
_params = pltpu.CompilerParams(needs_layout_passes=False)


def _t2_body(x_ref, w_ref, o_ref):
    o_ref[...] = jnp.dot(x_ref[...], w_ref[...],
                         preferred_element_type=jnp.float32)


def _compute_t2(x, wcat_t):
    rt = 1000
    return pl.pallas_call(
        _t2_body,
        grid=(NC, N // rt),
        in_specs=[
            pl.BlockSpec((rt, IN_F), lambda h, r: (r, 0)),
            pl.BlockSpec((IN_F, HALF), lambda h, r: (0, h)),
        ],
        out_specs=pl.BlockSpec((rt, HALF), lambda h, r: (h * (N // rt) + r, 0)),
        out_shape=jax.ShapeDtypeStruct((NC * N, HALF), jnp.float32),
    )(x, wcat_t)


def _scores_body(src_hbm, dst_hbm, t2_hbm, out_hbm,
                 sidx, didx, srowsA, drowsA, srowsB, drowsB, sch,
                 semAs, semAd, semBs, semBd):
    c = lax.axis_index("c")
    s = lax.axis_index("s")
    ibase = pl.multiple_of(c * E + s * EPW, 8)
    pltpu.sync_copy(src_hbm.at[pl.ds(ibase, EPW)], sidx)
    pltpu.sync_copy(dst_hbm.at[pl.ds(ibase, EPW)], didx)

    def start(i, srows, drows, sem_s, sem_d):
        sl = pl.ds(pl.multiple_of(i * CHUNK, 8), CHUNK)
        pltpu.async_copy(t2_hbm.at[sidx.at[sl]], srows, sem_s)
        pltpu.async_copy(t2_hbm.at[didx.at[sl]], drows, sem_d)

    def wait(i, srows, drows, sem_s, sem_d):
        sl = pl.ds(pl.multiple_of(i * CHUNK, 8), CHUNK)
        pltpu.make_async_copy(t2_hbm.at[sidx.at[sl]], srows, sem_s).wait()
        pltpu.make_async_copy(t2_hbm.at[didx.at[sl]], drows, sem_d).wait()

    def compute(i, srows, drows):
        def group(g, carry):
            iota = lax.broadcasted_iota(jnp.int32, (16,), 0)
            rid = iota + g * 16
            for h in range(HPC):
                acc = jnp.zeros((16,), jnp.float32)
                for cc in range(PER_HEAD):
                    col = h * PER_HEAD + ((cc + iota) & (PER_HEAD - 1))
                    sv = plsc.load_gather(srows, [rid, col])
                    dv = plsc.load_gather(drows, [rid, col])
                    acc = acc + sv * dv
                sch[pl.ds(h * EPW + i * CHUNK + g * 16, 16)] = acc
            return carry

        lax.fori_loop(0, CHUNK // 16, group, 0)

    start(0, srowsA, drowsA, semAs, semAd)

    def pair(ii, carry):
        i0 = ii * 2
        i1 = i0 + 1
        start(i1, srowsB, drowsB, semBs, semBd)
        wait(i0, srowsA, drowsA, semAs, semAd)
        compute(i0, srowsA, drowsA)
        start(i0 + 2, srowsA, drowsA, semAs, semAd)
        wait(i1, srowsB, drowsB, semBs, semBd)
        compute(i1, srowsB, drowsB)
        return carry

    lax.fori_loop(0, NPAIR, pair, 0)
    wait(NCHUNK - 1, srowsA, drowsA, semAs, semAd)
    compute(NCHUNK - 1, srowsA, drowsA)

    for h in range(HPC):
        hoff = pl.multiple_of((c * HPC + h) * E + s * EPW, 8)
        pltpu.sync_copy(sch.at[pl.ds(h * EPW, EPW)],
                        out_hbm.at[pl.ds(hoff, EPW)])


_sc_scores = functools.partial(
    pl.kernel,
    out_type=jax.ShapeDtypeStruct((NUM_HEADS * E,), jnp.float32),
    mesh=_mesh,
    compiler_params=_params,
    scratch_types=[
        pltpu.VMEM((EPW,), jnp.int32),
        pltpu.VMEM((EPW,), jnp.int32),
        pltpu.VMEM((CHUNK, HALF), jnp.float32),
        pltpu.VMEM((CHUNK, HALF), jnp.float32),
        pltpu.VMEM((CHUNK, HALF), jnp.float32),
        pltpu.VMEM((CHUNK, HALF), jnp.float32),
        pltpu.VMEM((HPC * EPW,), jnp.float32),
        pltpu.SemaphoreType.DMA,
        pltpu.SemaphoreType.DMA,
        pltpu.SemaphoreType.DMA,
        pltpu.SemaphoreType.DMA,
    ],
)(_scores_body)


def _softmax_body(s_ref, o_ref):
    sc = s_ref[...]
    m = jnp.max(sc, axis=1, keepdims=True)
    ex = jnp.exp(sc - m)
    z = jnp.sum(ex, axis=1, keepdims=True)
    o_ref[...] = ex / z


def _softmax(scores):
    return pl.pallas_call(
        _softmax_body,
        out_shape=jax.ShapeDtypeStruct((NUM_HEADS, E), jnp.float32),
    )(scores)


def _scatter_body(src_hbm, dst_hbm, t2_hbm, w_hbm, zeros_hbm, out_hbm,
                  sidx, didx, wchA, wchB, drowsA, drowsB, out_sh,
                  semA, semB, semSA, semSB):
    c = lax.axis_index("c")
    s = lax.axis_index("s")
    ebase = pl.multiple_of(s * EPW, 8)
    pltpu.sync_copy(src_hbm.at[pl.ds(ebase, EPW)], sidx)
    dbase = pl.multiple_of(c * E + s * EPW, 8)
    pltpu.sync_copy(dst_hbm.at[pl.ds(dbase, EPW)], didx)

    rbase = pl.multiple_of(s * ROWS_PER_SUB, 8)
    pltpu.sync_copy(zeros_hbm.at[pl.ds(rbase, ROWS_PER_SUB)],
                    out_sh.at[pl.ds(rbase, ROWS_PER_SUB)])
    plsc.subcore_barrier()

    def start(i, drows, wch, sem):
        sl = pl.ds(pl.multiple_of(i * CHUNK, 8), CHUNK)
        pltpu.async_copy(t2_hbm.at[didx.at[sl]], drows, sem)
        woff = pl.multiple_of((c * E + s * EPW + i * CHUNK) * HPC, 8)
        pltpu.async_copy(w_hbm.at[pl.ds(woff, CHUNK * HPC)], wch, sem)

    def wait(i, drows, wch, sem):
        sl = pl.ds(pl.multiple_of(i * CHUNK, 8), CHUNK)
        pltpu.make_async_copy(t2_hbm.at[didx.at[sl]], drows, sem).wait()
        woff = pl.multiple_of((c * E + s * EPW + i * CHUNK) * HPC, 8)
        pltpu.make_async_copy(w_hbm.at[pl.ds(woff, CHUNK * HPC)], wch,
                              sem).wait()

    def weight(i, drows, wch):
        @plsc.parallel_loop(0, CHUNK // 16, 1, unroll=2)
        def group(g):
            iota = lax.broadcasted_iota(jnp.int32, (16,), 0)
            rid = iota + g * 16
            rid4 = rid * HPC
            for h in range(HPC):
                wv = plsc.load_gather(wch, [rid4 + h])
                for b in range(PER_HEAD // 8):
                    cols = []
                    vals = []
                    for cc in range(b * 8, b * 8 + 8):
                        col = h * PER_HEAD + ((cc + iota) & (PER_HEAD - 1))
                        cols.append(col)
                        vals.append(plsc.load_gather(drows, [rid, col]))
                    for col, v in zip(cols, vals):
                        plsc.store_scatter(drows, [rid, col], v * wv)

    def start_scatter(i, drows, sem):
        for k in range(CHUNK // 16):
            idxv = sidx[pl.ds(i * CHUNK + k * 16, 16)]
            pltpu.async_copy(drows.at[pl.ds(k * 16, 16)],
                             out_sh.at[idxv], sem, add=True)

    def wait_scatter(i, drows, sem):
        for k in range(CHUNK // 16):
            idxv = sidx[pl.ds(i * CHUNK + k * 16, 16)]
            pltpu.make_async_copy(drows.at[pl.ds(k * 16, 16)],
                                  out_sh.at[idxv], sem).wait()

    start(0, drowsA, wchA, semA)

    def pair(ii, carry):
        i0 = ii * 2
        i1 = i0 + 1
        start(i1, drowsB, wchB, semB)
        wait(i0, drowsA, wchA, semA)
        weight(i0, drowsA, wchA)
        start_scatter(i0, drowsA, semSA)
        wait(i1, drowsB, wchB, semB)
        weight(i1, drowsB, wchB)
        start_scatter(i1, drowsB, semSB)
        wait_scatter(i0, drowsA, semSA)
        start(i0 + 2, drowsA, wchA, semA)
        wait_scatter(i1, drowsB, semSB)
        return carry

    lax.fori_loop(0, NPAIR, pair, 0)
    i_last = NCHUNK - 1
    wait(i_last, drowsA, wchA, semA)
    weight(i_last, drowsA, wchA)
    start_scatter(i_last, drowsA, semSA)
    wait_scatter(i_last, drowsA, semSA)

    plsc.subcore_barrier()
    obase = pl.multiple_of(c * N_PAD + s * ROWS_PER_SUB, 8)
    pltpu.sync_copy(out_sh.at[pl.ds(rbase, ROWS_PER_SUB)],
                    out_hbm.at[pl.ds(obase, ROWS_PER_SUB)])


_sc_scatter = functools.partial(
    pl.kernel,
    out_type=jax.ShapeDtypeStruct((NC * N_PAD, HALF), jnp.float32),
    mesh=_mesh,
    compiler_params=_params,
    scratch_types=[
        pltpu.VMEM((EPW,), jnp.int32),
        pltpu.VMEM((EPW,), jnp.int32),
        pltpu.VMEM((CHUNK * HPC,), jnp.float32),
        pltpu.VMEM((CHUNK * HPC,), jnp.float32),
        pltpu.VMEM((CHUNK, HALF), jnp.float32),
        pltpu.VMEM((CHUNK, HALF), jnp.float32),
        pltpu.VMEM_SHARED((N_PAD, HALF), jnp.float32),
        pltpu.SemaphoreType.DMA,
        pltpu.SemaphoreType.DMA,
        pltpu.SemaphoreType.DMA,
        pltpu.SemaphoreType.DMA,
    ],
)(_scatter_body)


def _merge_body(a0_ref, a1_ref, w0_ref, w1_ref, b_ref, o_ref):
    o_ref[...] = (jnp.dot(a0_ref[...], w0_ref[...],
                          preferred_element_type=jnp.float32)
                  + jnp.dot(a1_ref[...], w1_ref[...],
                            preferred_element_type=jnp.float32)
                  + b_ref[...])


def _merge(o0, o1, w0, w1, bm2):
    rt = 1000
    return pl.pallas_call(
        _merge_body,
        grid=(N // rt,),
        in_specs=[
            pl.BlockSpec((rt, HALF), lambda r: (r, 0)),
            pl.BlockSpec((rt, HALF), lambda r: (r, 0)),
            pl.BlockSpec((HALF, OUT_F), lambda r: (0, 0)),
            pl.BlockSpec((HALF, OUT_F), lambda r: (0, 0)),
            pl.BlockSpec((1, OUT_F), lambda r: (0, 0)),
        ],
        out_specs=pl.BlockSpec((rt, OUT_F), lambda r: (r, 0)),
        out_shape=jax.ShapeDtypeStruct((N, OUT_F), jnp.float32),
    )(o0, o1, w0, w1, bm2)


def kernel(node_features, edge_index, W, Wm, bm):
    src = edge_index[0]
    dst = edge_index[1]
    src2 = jnp.concatenate([src, src + N])
    dst2 = jnp.concatenate([dst, dst + N])
    wcat_t = W.reshape(OUT_F, IN_F).T
    t2 = _compute_t2(node_features, wcat_t)
    scores = _sc_scores(src2, dst2, t2)
    w = _softmax(scores.reshape(NUM_HEADS, E))
    zeros = jnp.zeros((N_PAD, HALF), jnp.float32)
    w_em = w.reshape(NC, HPC, E).transpose(0, 2, 1).reshape(-1)
    out2 = _sc_scatter(src, dst2, t2, w_em, zeros)
    w0 = Wm[:, :HALF].T
    w1 = Wm[:, HALF:].T
    return _merge(out2[:N], out2[N_PAD:N_PAD + N], w0, w1,
                  bm.reshape(1, OUT_F))


# speedup vs baseline: 1.4718x; 1.1453x over previous
import functools

import jax
import jax.numpy as jnp
from jax import lax
from jax.experimental import pallas as pl
from jax.experimental.pallas import tpu as pltpu
from jax.experimental.pallas import tpu_sc as plsc

NUM_HEADS = 8
IN_F = 256
OUT_F = 256
PER_HEAD = 32
N = 10000
E = 160000
HALF = 128
HPC = 4

NC = 2
NS = 16
EPW = E // NS
CHUNK = 80
NCHUNK = EPW // CHUNK
NPAIR = (NCHUNK - 1) // 2
N_PAD = 10240
ROWS_PER_SUB = N_PAD // NS
MZ_PER_SUB = 2 * HPC * 16

_mesh = plsc.VectorSubcoreMesh(core_axis_name="c", subcore_axis_name="s")
_params = pltpu.CompilerParams(needs_layout_passes=False)


def _t2_body(x_ref, w_ref, o_ref):
    o_ref[...] = jnp.dot(x_ref[...], w_ref[...],
                         preferred_element_type=jnp.float32)


def _compute_t2(x, wcat_t):
    rt = 1000
    return pl.pallas_call(
        _t2_body,
        grid=(NC, N // rt),
        in_specs=[
            pl.BlockSpec((rt, IN_F), lambda h, r: (r, 0)),
            pl.BlockSpec((IN_F, HALF), lambda h, r: (0, h)),
        ],
        out_specs=pl.BlockSpec((rt, HALF), lambda h, r: (h * (N // rt) + r, 0)),
        out_shape=jax.ShapeDtypeStruct((NC * N, HALF), jnp.float32),
    )(x, wcat_t)


def _scores_body(src_hbm, dst_hbm, t2_hbm, out_hbm, mz_hbm,
                 sidx, didx, srowsA, drowsA, srowsB, drowsB, sch, mzbuf,
                 semAs, semAd, semBs, semBd):
    c = lax.axis_index("c")
    s = lax.axis_index("s")
    ibase = pl.multiple_of(c * E + s * EPW, 8)
    pltpu.sync_copy(src_hbm.at[pl.ds(ibase, EPW)], sidx)
    pltpu.sync_copy(dst_hbm.at[pl.ds(ibase, EPW)], didx)

    def start(i, srows, drows, sem_s, sem_d):
        sl = pl.ds(pl.multiple_of(i * CHUNK, 8), CHUNK)
        pltpu.async_copy(t2_hbm.at[sidx.at[sl]], srows, sem_s)
        pltpu.async_copy(t2_hbm.at[didx.at[sl]], drows, sem_d)

    def wait(i, srows, drows, sem_s, sem_d):
        sl = pl.ds(pl.multiple_of(i * CHUNK, 8), CHUNK)
        pltpu.make_async_copy(t2_hbm.at[sidx.at[sl]], srows, sem_s).wait()
        pltpu.make_async_copy(t2_hbm.at[didx.at[sl]], drows, sem_d).wait()

    def compute(i, srows, drows):
        def group(g, carry):
            iota = lax.broadcasted_iota(jnp.int32, (16,), 0)
            rid = iota + g * 16
            for h in range(HPC):
                acc = jnp.zeros((16,), jnp.float32)
                for cc in range(PER_HEAD):
                    col = h * PER_HEAD + ((cc + iota) & (PER_HEAD - 1))
                    sv = plsc.load_gather(srows, [rid, col])
                    dv = plsc.load_gather(drows, [rid, col])
                    acc = acc + sv * dv
                sch[pl.ds(h * EPW + i * CHUNK + g * 16, 16)] = acc
            return carry

        lax.fori_loop(0, CHUNK // 16, group, 0)

        def mzupd(g, carry):
            for h in range(HPC):
                acc = sch[pl.ds(h * EPW + i * CHUNK + g * 16, 16)]
                m_old = mzbuf[pl.ds(h * 16, 16)]
                z_old = mzbuf[pl.ds((HPC + h) * 16, 16)]
                m_new = jnp.maximum(m_old, acc)
                mzbuf[pl.ds(h * 16, 16)] = m_new
                mzbuf[pl.ds((HPC + h) * 16, 16)] = (
                    z_old * jnp.exp(m_old - m_new) + jnp.exp(acc - m_new))
            return carry

        lax.fori_loop(0, CHUNK // 16, mzupd, 0)

    for h in range(HPC):
        mzbuf[pl.ds(h * 16, 16)] = jnp.full((16,), -1e30, jnp.float32)
        mzbuf[pl.ds((HPC + h) * 16, 16)] = jnp.zeros((16,), jnp.float32)

    start(0, srowsA, drowsA, semAs, semAd)

    def pair(ii, carry):
        i0 = ii * 2
        i1 = i0 + 1
        start(i1, srowsB, drowsB, semBs, semBd)
        wait(i0, srowsA, drowsA, semAs, semAd)
        compute(i0, srowsA, drowsA)
        start(i0 + 2, srowsA, drowsA, semAs, semAd)
        wait(i1, srowsB, drowsB, semBs, semBd)
        compute(i1, srowsB, drowsB)
        return carry

    lax.fori_loop(0, NPAIR, pair, 0)
    wait(NCHUNK - 1, srowsA, drowsA, semAs, semAd)
    compute(NCHUNK - 1, srowsA, drowsA)

    mzoff = pl.multiple_of((c * NS + s) * MZ_PER_SUB, 8)
    pltpu.sync_copy(mzbuf, mz_hbm.at[pl.ds(mzoff, MZ_PER_SUB)])
    for h in range(HPC):
        hoff = pl.multiple_of((c * HPC + h) * E + s * EPW, 8)
        pltpu.sync_copy(sch.at[pl.ds(h * EPW, EPW)],
                        out_hbm.at[pl.ds(hoff, EPW)])


_sc_scores = functools.partial(
    pl.kernel,
    out_type=[
        jax.ShapeDtypeStruct((NUM_HEADS * E,), jnp.float32),
        jax.ShapeDtypeStruct((NC * NS * MZ_PER_SUB,), jnp.float32),
    ],
    mesh=_mesh,
    compiler_params=_params,
    scratch_types=[
        pltpu.VMEM((EPW,), jnp.int32),
        pltpu.VMEM((EPW,), jnp.int32),
        pltpu.VMEM((CHUNK, HALF), jnp.float32),
        pltpu.VMEM((CHUNK, HALF), jnp.float32),
        pltpu.VMEM((CHUNK, HALF), jnp.float32),
        pltpu.VMEM((CHUNK, HALF), jnp.float32),
        pltpu.VMEM((EPW * HPC,), jnp.float32),
        pltpu.VMEM((MZ_PER_SUB,), jnp.float32),
        pltpu.SemaphoreType.DMA,
        pltpu.SemaphoreType.DMA,
        pltpu.SemaphoreType.DMA,
        pltpu.SemaphoreType.DMA,
    ],
)(_scores_body)


def _scatter_body(src_hbm, dst_hbm, t2_hbm, sc_hbm, mz_hbm, zeros_hbm,
                  out_hbm,
                  sidx, didx, scoA, scoB, drowsA, drowsB, mzb, mzc, out_sh,
                  semA, semB, semSA, semSB):
    c = lax.axis_index("c")
    s = lax.axis_index("s")
    ebase = pl.multiple_of(s * EPW, 8)
    pltpu.sync_copy(src_hbm.at[pl.ds(ebase, EPW)], sidx)
    dbase = pl.multiple_of(c * E + s * EPW, 8)
    pltpu.sync_copy(dst_hbm.at[pl.ds(dbase, EPW)], didx)
    pltpu.sync_copy(mz_hbm.at[pl.ds(pl.multiple_of(c * NS * MZ_PER_SUB, 8),
                                    NS * MZ_PER_SUB)], mzb)

    iota0 = lax.broadcasted_iota(jnp.int32, (16,), 0)
    for h in range(HPC):
        mv = mzb[pl.ds(h * 16, 16)]
        for t in range(1, NS):
            mv = jnp.maximum(mv, mzb[pl.ds(t * MZ_PER_SUB + h * 16, 16)])
        mzc[pl.ds(h * 16, 16)] = mv
        for sh in (8, 4, 2, 1):
            v = mzc[pl.ds(h * 16, 16)]
            vs = plsc.load_gather(mzc, [h * 16 + ((iota0 + sh) & 15)])
            mzc[pl.ds(h * 16, 16)] = jnp.maximum(v, vs)
        mh = mzc[pl.ds(h * 16, 16)]
        zv = jnp.zeros((16,), jnp.float32)
        for t in range(NS):
            mt = mzb[pl.ds(t * MZ_PER_SUB + h * 16, 16)]
            zt = mzb[pl.ds(t * MZ_PER_SUB + (HPC + h) * 16, 16)]
            zv = zv + zt * jnp.exp(mt - mh)
        zoff = (HPC + h) * 16
        mzc[pl.ds(zoff, 16)] = zv
        for sh in (8, 4, 2, 1):
            v = mzc[pl.ds(zoff, 16)]
            vs = plsc.load_gather(mzc, [zoff + ((iota0 + sh) & 15)])
            mzc[pl.ds(zoff, 16)] = v + vs
        mzc[pl.ds(zoff, 16)] = (jnp.ones((16,), jnp.float32)
                                / mzc[pl.ds(zoff, 16)])

    rbase = pl.multiple_of(s * ROWS_PER_SUB, 8)
    pltpu.sync_copy(zeros_hbm.at[pl.ds(rbase, ROWS_PER_SUB)],
                    out_sh.at[pl.ds(rbase, ROWS_PER_SUB)])
    plsc.subcore_barrier()

    def start(i, drows, sco, sem):
        sl = pl.ds(pl.multiple_of(i * CHUNK, 8), CHUNK)
        pltpu.async_copy(t2_hbm.at[didx.at[sl]], drows, sem)
        for h in range(HPC):
            soff = pl.multiple_of((c * HPC + h) * E + s * EPW + i * CHUNK, 8)
            pltpu.async_copy(sc_hbm.at[pl.ds(soff, CHUNK)],
                             sco.at[pl.ds(h * CHUNK, CHUNK)], sem)

    def wait(i, drows, sco, sem):
        sl = pl.ds(pl.multiple_of(i * CHUNK, 8), CHUNK)
        pltpu.make_async_copy(t2_hbm.at[didx.at[sl]], drows, sem).wait()
        for h in range(HPC):
            soff = pl.multiple_of((c * HPC + h) * E + s * EPW + i * CHUNK, 8)
            pltpu.make_async_copy(sc_hbm.at[pl.ds(soff, CHUNK)],
                                  sco.at[pl.ds(h * CHUNK, CHUNK)], sem).wait()

    def weight(i, drows, sco):
        @plsc.parallel_loop(0, CHUNK // 16, 1, unroll=1)
        def group(g):
            iota = lax.broadcasted_iota(jnp.int32, (16,), 0)
            rid = iota + g * 16
            for h in range(HPC):
                sv = sco[pl.ds(h * CHUNK + g * 16, 16)]
                wv = (jnp.exp(sv - mzc[pl.ds(h * 16, 16)])
                      * mzc[pl.ds((HPC + h) * 16, 16)])
                for b in range(PER_HEAD // 8):
                    cols = []
                    vals = []
                    for cc in range(b * 8, b * 8 + 8):
                        col = h * PER_HEAD + ((cc + iota) & (PER_HEAD - 1))
                        cols.append(col)
                        vals.append(plsc.load_gather(drows, [rid, col]))
                    for col, v in zip(cols, vals):
                        plsc.store_scatter(drows, [rid, col], v * wv)

    def start_scatter(i, drows, sem):
        for k in range(CHUNK // 16):
            idxv = sidx[pl.ds(i * CHUNK + k * 16, 16)]
            pltpu.async_copy(drows.at[pl.ds(k * 16, 16)],
                             out_sh.at[idxv], sem, add=True)

    def wait_scatter(i, drows, sem):
        for k in range(CHUNK // 16):
            idxv = sidx[pl.ds(i * CHUNK + k * 16, 16)]
            pltpu.make_async_copy(drows.at[pl.ds(k * 16, 16)],
                                  out_sh.at[idxv], sem).wait()

    start(0, drowsA, scoA, semA)

    def pair(ii, carry):
        i0 = ii * 2
        i1 = i0 + 1
        start(i1, drowsB, scoB, semB)
        wait(i0, drowsA, scoA, semA)
        weight(i0, drowsA, scoA)
        start_scatter(i0, drowsA, semSA)
        wait(i1, drowsB, scoB, semB)
        weight(i1, drowsB, scoB)
        start_scatter(i1, drowsB, semSB)
        wait_scatter(i0, drowsA, semSA)
        start(i0 + 2, drowsA, scoA, semA)
        wait_scatter(i1, drowsB, semSB)
        return carry

    lax.fori_loop(0, NPAIR, pair, 0)
    i_last = NCHUNK - 1
    wait(i_last, drowsA, scoA, semA)
    weight(i_last, drowsA, scoA)
    start_scatter(i_last, drowsA, semSA)
    wait_scatter(i_last, drowsA, semSA)

    plsc.subcore_barrier()
    obase = pl.multiple_of(c * N_PAD + s * ROWS_PER_SUB, 8)
    pltpu.sync_copy(out_sh.at[pl.ds(rbase, ROWS_PER_SUB)],
                    out_hbm.at[pl.ds(obase, ROWS_PER_SUB)])


_sc_scatter = functools.partial(
    pl.kernel,
    out_type=jax.ShapeDtypeStruct((NC * N_PAD, HALF), jnp.float32),
    mesh=_mesh,
    compiler_params=_params,
    scratch_types=[
        pltpu.VMEM((EPW,), jnp.int32),
        pltpu.VMEM((EPW,), jnp.int32),
        pltpu.VMEM((CHUNK * HPC,), jnp.float32),
        pltpu.VMEM((CHUNK * HPC,), jnp.float32),
        pltpu.VMEM((CHUNK, HALF), jnp.float32),
        pltpu.VMEM((CHUNK, HALF), jnp.float32),
        pltpu.VMEM((NS * MZ_PER_SUB,), jnp.float32),
        pltpu.VMEM((MZ_PER_SUB,), jnp.float32),
        pltpu.VMEM_SHARED((N_PAD, HALF), jnp.float32),
        pltpu.SemaphoreType.DMA,
        pltpu.SemaphoreType.DMA,
        pltpu.SemaphoreType.DMA,
        pltpu.SemaphoreType.DMA,
    ],
)(_scatter_body)


def _merge_body(a0_ref, a1_ref, w0_ref, w1_ref, b_ref, o_ref):
    o_ref[...] = (jnp.dot(a0_ref[...], w0_ref[...],
                          preferred_element_type=jnp.float32)
                  + jnp.dot(a1_ref[...], w1_ref[...],
                            preferred_element_type=jnp.float32)
                  + b_ref[...])


def _merge(o0, o1, w0, w1, bm2):
    rt = 1000
    return pl.pallas_call(
        _merge_body,
        grid=(N // rt,),
        in_specs=[
            pl.BlockSpec((rt, HALF), lambda r: (r, 0)),
            pl.BlockSpec((rt, HALF), lambda r: (r, 0)),
            pl.BlockSpec((HALF, OUT_F), lambda r: (0, 0)),
            pl.BlockSpec((HALF, OUT_F), lambda r: (0, 0)),
            pl.BlockSpec((1, OUT_F), lambda r: (0, 0)),
        ],
        out_specs=pl.BlockSpec((rt, OUT_F), lambda r: (r, 0)),
        out_shape=jax.ShapeDtypeStruct((N, OUT_F), jnp.float32),
    )(o0, o1, w0, w1, bm2)


def kernel(node_features, edge_index, W, Wm, bm):
    src = edge_index[0]
    dst = edge_index[1]
    src2 = jnp.concatenate([src, src + N])
    dst2 = jnp.concatenate([dst, dst + N])
    wcat_t = W.reshape(OUT_F, IN_F).T
    t2 = _compute_t2(node_features, wcat_t)
    scores_em, mz = _sc_scores(src2, dst2, t2)
    zeros = jnp.zeros((N_PAD, HALF), jnp.float32)
    out2 = _sc_scatter(src, dst2, t2, scores_em, mz, zeros)
    w0 = Wm[:, :HALF].T
    w1 = Wm[:, HALF:].T
    return _merge(out2[:N], out2[N_PAD:N_PAD + N], w0, w1,
                  bm.reshape(1, OUT_F))


# speedup vs baseline: 1.7623x; 1.1973x over previous
import functools

import jax
import jax.numpy as jnp
from jax import lax
from jax.experimental import pallas as pl
from jax.experimental.pallas import tpu as pltpu
from jax.experimental.pallas import tpu_sc as plsc

NUM_HEADS = 8
IN_F = 256
OUT_F = 256
PER_HEAD = 32
N = 10000
E = 160000
HALF = 128
HPC = 4

NC = 2
NS = 16
EPW = E // NS
CHUNK = 80
NCHUNK = EPW // CHUNK
NPAIR = (NCHUNK - 1) // 2
N_PAD = 10240
ROWS_PER_SUB = N_PAD // NS
MZ_PER_SUB = 2 * HPC * 16

_mesh = plsc.VectorSubcoreMesh(core_axis_name="c", subcore_axis_name="s")
_params = pltpu.CompilerParams(needs_layout_passes=False)


def _t2_body(x_ref, w_ref, o_ref):
    o_ref[...] = jnp.dot(x_ref[...], w_ref[...],
                         preferred_element_type=jnp.float32)


def _compute_t2(x, wcat_t):
    rt = 1000
    return pl.pallas_call(
        _t2_body,
        grid=(NC, N // rt),
        in_specs=[
            pl.BlockSpec((rt, IN_F), lambda h, r: (r, 0)),
            pl.BlockSpec((IN_F, HALF), lambda h, r: (0, h)),
        ],
        out_specs=pl.BlockSpec((rt, HALF), lambda h, r: (h * (N // rt) + r, 0)),
        out_shape=jax.ShapeDtypeStruct((NC * N, HALF), jnp.float32),
    )(x, wcat_t)


def _scores_body(src_hbm, dst_hbm, t2_hbm, out_hbm, mz_hbm,
                 sidx, didx, srowsA, drowsA, srowsB, drowsB, sch, mzbuf,
                 semAs, semAd, semBs, semBd):
    c = lax.axis_index("c")
    s = lax.axis_index("s")
    ibase = pl.multiple_of(c * E + s * EPW, 8)
    pltpu.sync_copy(src_hbm.at[pl.ds(ibase, EPW)], sidx)
    pltpu.sync_copy(dst_hbm.at[pl.ds(ibase, EPW)], didx)

    def start(i, srows, drows, sem_s, sem_d):
        sl = pl.ds(pl.multiple_of(i * CHUNK, 8), CHUNK)
        pltpu.async_copy(t2_hbm.at[sidx.at[sl]], srows, sem_s)
        pltpu.async_copy(t2_hbm.at[didx.at[sl]], drows, sem_d)

    def wait(i, srows, drows, sem_s, sem_d):
        sl = pl.ds(pl.multiple_of(i * CHUNK, 8), CHUNK)
        pltpu.make_async_copy(t2_hbm.at[sidx.at[sl]], srows, sem_s).wait()
        pltpu.make_async_copy(t2_hbm.at[didx.at[sl]], drows, sem_d).wait()

    def compute(i, srows, drows):
        def group(g, carry):
            iota = lax.broadcasted_iota(jnp.int32, (16,), 0)
            rid = iota + g * 16
            for h in range(HPC):
                acc = jnp.zeros((16,), jnp.float32)
                for cc in range(PER_HEAD):
                    col = h * PER_HEAD + ((cc + iota) & (PER_HEAD - 1))
                    sv = plsc.load_gather(srows, [rid, col])
                    dv = plsc.load_gather(drows, [rid, col])
                    acc = acc + sv * dv
                sch[pl.ds(h * EPW + i * CHUNK + g * 16, 16)] = acc
            return carry

        lax.fori_loop(0, CHUNK // 16, group, 0)

    start(0, srowsA, drowsA, semAs, semAd)

    def pair(ii, carry):
        i0 = ii * 2
        i1 = i0 + 1
        start(i1, srowsB, drowsB, semBs, semBd)
        wait(i0, srowsA, drowsA, semAs, semAd)
        compute(i0, srowsA, drowsA)
        start(i0 + 2, srowsA, drowsA, semAs, semAd)
        wait(i1, srowsB, drowsB, semBs, semBd)
        compute(i1, srowsB, drowsB)
        return carry

    lax.fori_loop(0, NPAIR, pair, 0)
    wait(NCHUNK - 1, srowsA, drowsA, semAs, semAd)
    compute(NCHUNK - 1, srowsA, drowsA)

    for h in range(HPC):
        mzbuf[pl.ds(h * 16, 16)] = jnp.full((16,), -1e30, jnp.float32)
        mzbuf[pl.ds((HPC + h) * 16, 16)] = jnp.zeros((16,), jnp.float32)

    def mzsweep(g, carry):
        for h in range(HPC):
            acc = sch[pl.ds(h * EPW + g * 16, 16)]
            m_old = mzbuf[pl.ds(h * 16, 16)]
            z_old = mzbuf[pl.ds((HPC + h) * 16, 16)]
            m_new = jnp.maximum(m_old, acc)
            mzbuf[pl.ds(h * 16, 16)] = m_new
            mzbuf[pl.ds((HPC + h) * 16, 16)] = (
                z_old * jnp.exp(m_old - m_new) + jnp.exp(acc - m_new))
        return carry

    lax.fori_loop(0, EPW // 16, mzsweep, 0)

    mzoff = pl.multiple_of((c * NS + s) * MZ_PER_SUB, 8)
    pltpu.sync_copy(mzbuf, mz_hbm.at[pl.ds(mzoff, MZ_PER_SUB)])
    for h in range(HPC):
        hoff = pl.multiple_of((c * HPC + h) * E + s * EPW, 8)
        pltpu.sync_copy(sch.at[pl.ds(h * EPW, EPW)],
                        out_hbm.at[pl.ds(hoff, EPW)])


_sc_scores = functools.partial(
    pl.kernel,
    out_type=[
        jax.ShapeDtypeStruct((NUM_HEADS * E,), jnp.float32),
        jax.ShapeDtypeStruct((NC * NS * MZ_PER_SUB,), jnp.float32),
    ],
    mesh=_mesh,
    compiler_params=_params,
    scratch_types=[
        pltpu.VMEM((EPW,), jnp.int32),
        pltpu.VMEM((EPW,), jnp.int32),
        pltpu.VMEM((CHUNK, HALF), jnp.float32),
        pltpu.VMEM((CHUNK, HALF), jnp.float32),
        pltpu.VMEM((CHUNK, HALF), jnp.float32),
        pltpu.VMEM((CHUNK, HALF), jnp.float32),
        pltpu.VMEM((EPW * HPC,), jnp.float32),
        pltpu.VMEM((MZ_PER_SUB,), jnp.float32),
        pltpu.SemaphoreType.DMA,
        pltpu.SemaphoreType.DMA,
        pltpu.SemaphoreType.DMA,
        pltpu.SemaphoreType.DMA,
    ],
)(_scores_body)


def _scatter_body(src_hbm, dst_hbm, t2_hbm, sc_hbm, mz_hbm, zeros_hbm,
                  out_hbm,
                  sidx, didx, scoA, scoB, drowsA, drowsB, mzb, mzc, out_sh,
                  semA, semB, semSA, semSB):
    c = lax.axis_index("c")
    s = lax.axis_index("s")
    ebase = pl.multiple_of(s * EPW, 8)
    pltpu.sync_copy(src_hbm.at[pl.ds(ebase, EPW)], sidx)
    dbase = pl.multiple_of(c * E + s * EPW, 8)
    pltpu.sync_copy(dst_hbm.at[pl.ds(dbase, EPW)], didx)
    pltpu.sync_copy(mz_hbm.at[pl.ds(pl.multiple_of(c * NS * MZ_PER_SUB, 8),
                                    NS * MZ_PER_SUB)], mzb)

    iota0 = lax.broadcasted_iota(jnp.int32, (16,), 0)
    for h in range(HPC):
        mv = mzb[pl.ds(h * 16, 16)]
        for t in range(1, NS):
            mv = jnp.maximum(mv, mzb[pl.ds(t * MZ_PER_SUB + h * 16, 16)])
        mzc[pl.ds(h * 16, 16)] = mv
        for sh in (8, 4, 2, 1):
            v = mzc[pl.ds(h * 16, 16)]
            vs = plsc.load_gather(mzc, [h * 16 + ((iota0 + sh) & 15)])
            mzc[pl.ds(h * 16, 16)] = jnp.maximum(v, vs)
        mh = mzc[pl.ds(h * 16, 16)]
        zv = jnp.zeros((16,), jnp.float32)
        for t in range(NS):
            mt = mzb[pl.ds(t * MZ_PER_SUB + h * 16, 16)]
            zt = mzb[pl.ds(t * MZ_PER_SUB + (HPC + h) * 16, 16)]
            zv = zv + zt * jnp.exp(mt - mh)
        zoff = (HPC + h) * 16
        mzc[pl.ds(zoff, 16)] = zv
        for sh in (8, 4, 2, 1):
            v = mzc[pl.ds(zoff, 16)]
            vs = plsc.load_gather(mzc, [zoff + ((iota0 + sh) & 15)])
            mzc[pl.ds(zoff, 16)] = v + vs
        mzc[pl.ds(zoff, 16)] = (jnp.ones((16,), jnp.float32)
                                / mzc[pl.ds(zoff, 16)])

    rbase = pl.multiple_of(s * ROWS_PER_SUB, 8)
    pltpu.sync_copy(zeros_hbm.at[pl.ds(rbase, ROWS_PER_SUB)],
                    out_sh.at[pl.ds(rbase, ROWS_PER_SUB)])
    plsc.subcore_barrier()

    def start(i, drows, sco, sem):
        sl = pl.ds(pl.multiple_of(i * CHUNK, 8), CHUNK)
        pltpu.async_copy(t2_hbm.at[didx.at[sl]], drows, sem)
        for h in range(HPC):
            soff = pl.multiple_of((c * HPC + h) * E + s * EPW + i * CHUNK, 8)
            pltpu.async_copy(sc_hbm.at[pl.ds(soff, CHUNK)],
                             sco.at[pl.ds(h * CHUNK, CHUNK)], sem)

    def wait(i, drows, sco, sem):
        sl = pl.ds(pl.multiple_of(i * CHUNK, 8), CHUNK)
        pltpu.make_async_copy(t2_hbm.at[didx.at[sl]], drows, sem).wait()
        for h in range(HPC):
            soff = pl.multiple_of((c * HPC + h) * E + s * EPW + i * CHUNK, 8)
            pltpu.make_async_copy(sc_hbm.at[pl.ds(soff, CHUNK)],
                                  sco.at[pl.ds(h * CHUNK, CHUNK)], sem).wait()

    def weight(i, drows, sco):
        @plsc.parallel_loop(0, CHUNK // 16, 1, unroll=1)
        def group(g):
            iota = lax.broadcasted_iota(jnp.int32, (16,), 0)
            rid = iota + g * 16
            for h in range(HPC):
                sv = sco[pl.ds(h * CHUNK + g * 16, 16)]
                wv = (jnp.exp(sv - mzc[pl.ds(h * 16, 16)])
                      * mzc[pl.ds((HPC + h) * 16, 16)])
                for b in range(PER_HEAD // 16):
                    cols = []
                    vals = []
                    for cc in range(b * 16, b * 16 + 16):
                        col = h * PER_HEAD + ((cc + iota) & (PER_HEAD - 1))
                        cols.append(col)
                        vals.append(plsc.load_gather(drows, [rid, col]))
                    for col, v in zip(cols, vals):
                        plsc.store_scatter(drows, [rid, col], v * wv)

    def start_scatter(i, drows, sem):
        for k in range(CHUNK // 16):
            idxv = sidx[pl.ds(i * CHUNK + k * 16, 16)]
            pltpu.async_copy(drows.at[pl.ds(k * 16, 16)],
                             out_sh.at[idxv], sem, add=True)

    def wait_scatter(i, drows, sem):
        for k in range(CHUNK // 16):
            idxv = sidx[pl.ds(i * CHUNK + k * 16, 16)]
            pltpu.make_async_copy(drows.at[pl.ds(k * 16, 16)],
                                  out_sh.at[idxv], sem).wait()

    start(0, drowsA, scoA, semA)

    def pair(ii, carry):
        i0 = ii * 2
        i1 = i0 + 1
        start(i1, drowsB, scoB, semB)
        wait(i0, drowsA, scoA, semA)
        weight(i0, drowsA, scoA)
        start_scatter(i0, drowsA, semSA)
        wait(i1, drowsB, scoB, semB)
        weight(i1, drowsB, scoB)
        start_scatter(i1, drowsB, semSB)
        wait_scatter(i0, drowsA, semSA)
        start(i0 + 2, drowsA, scoA, semA)
        wait_scatter(i1, drowsB, semSB)
        return carry

    lax.fori_loop(0, NPAIR, pair, 0)
    i_last = NCHUNK - 1
    wait(i_last, drowsA, scoA, semA)
    weight(i_last, drowsA, scoA)
    start_scatter(i_last, drowsA, semSA)
    wait_scatter(i_last, drowsA, semSA)

    plsc.subcore_barrier()
    obase = pl.multiple_of(c * N_PAD + s * ROWS_PER_SUB, 8)
    pltpu.sync_copy(out_sh.at[pl.ds(rbase, ROWS_PER_SUB)],
                    out_hbm.at[pl.ds(obase, ROWS_PER_SUB)])


_sc_scatter = functools.partial(
    pl.kernel,
    out_type=jax.ShapeDtypeStruct((NC * N_PAD, HALF), jnp.float32),
    mesh=_mesh,
    compiler_params=_params,
    scratch_types=[
        pltpu.VMEM((EPW,), jnp.int32),
        pltpu.VMEM((EPW,), jnp.int32),
        pltpu.VMEM((CHUNK * HPC,), jnp.float32),
        pltpu.VMEM((CHUNK * HPC,), jnp.float32),
        pltpu.VMEM((CHUNK, HALF), jnp.float32),
        pltpu.VMEM((CHUNK, HALF), jnp.float32),
        pltpu.VMEM((NS * MZ_PER_SUB,), jnp.float32),
        pltpu.VMEM((MZ_PER_SUB,), jnp.float32),
        pltpu.VMEM_SHARED((N_PAD, HALF), jnp.float32),
        pltpu.SemaphoreType.DMA,
        pltpu.SemaphoreType.DMA,
        pltpu.SemaphoreType.DMA,
        pltpu.SemaphoreType.DMA,
    ],
)(_scatter_body)


def _merge_body(a0_ref, a1_ref, w0_ref, w1_ref, b_ref, o_ref):
    o_ref[...] = (jnp.dot(a0_ref[...], w0_ref[...],
                          preferred_element_type=jnp.float32)
                  + jnp.dot(a1_ref[...], w1_ref[...],
                            preferred_element_type=jnp.float32)
                  + b_ref[...])


def _merge(o0, o1, w0, w1, bm2):
    rt = 1000
    return pl.pallas_call(
        _merge_body,
        grid=(N // rt,),
        in_specs=[
            pl.BlockSpec((rt, HALF), lambda r: (r, 0)),
            pl.BlockSpec((rt, HALF), lambda r: (r, 0)),
            pl.BlockSpec((HALF, OUT_F), lambda r: (0, 0)),
            pl.BlockSpec((HALF, OUT_F), lambda r: (0, 0)),
            pl.BlockSpec((1, OUT_F), lambda r: (0, 0)),
        ],
        out_specs=pl.BlockSpec((rt, OUT_F), lambda r: (r, 0)),
        out_shape=jax.ShapeDtypeStruct((N, OUT_F), jnp.float32),
    )(o0, o1, w0, w1, bm2)


def kernel(node_features, edge_index, W, Wm, bm):
    src = edge_index[0]
    dst = edge_index[1]
    src2 = jnp.concatenate([src, src + N])
    dst2 = jnp.concatenate([dst, dst + N])
    wcat_t = W.reshape(OUT_F, IN_F).T
    t2 = _compute_t2(node_features, wcat_t)
    scores_em, mz = _sc_scores(src2, dst2, t2)
    zeros = jnp.zeros((N_PAD, HALF), jnp.float32)
    out2 = _sc_scatter(src, dst2, t2, scores_em, mz, zeros)
    w0 = Wm[:, :HALF].T
    w1 = Wm[:, HALF:].T
    return _merge(out2[:N], out2[N_PAD:N_PAD + N], w0, w1,
                  bm.reshape(1, OUT_F))
